# baseline (speedup 1.0000x reference)
import functools

import jax
import jax.numpy as jnp
from jax import lax
from jax.experimental import pallas as pl
from jax.experimental.pallas import tpu as pltpu
from jax.experimental.pallas import tpu_sc as plsc

B = 16384
DU = 64
DG = 32
PW = 128
NC = 2
NS = 16
NW = NC * NS
BPW = B // NW
CH = 256
NCHK = BPW // CH


def _sc_gather_body(uidx_hbm, iidx_hbm, ut_hbm, it_hbm,
                    uo_hbm, io_hbm,
                    uidx_v, iidx_v, buf0, buf1,
                    g0, g1, w0, w1):
    wid = lax.axis_index("s") * NC + lax.axis_index("c")
    base = wid * BPW
    pltpu.sync_copy(uidx_hbm.at[pl.ds(base, BPW)], uidx_v)
    pltpu.sync_copy(iidx_hbm.at[pl.ds(base, BPW)], iidx_v)

    bufs = (buf0, buf1)
    gsems = (g0, g1)
    wsems = (w0, w1)
    writes = [None, None]
    for tbl, idx_v, out in ((ut_hbm, uidx_v, uo_hbm), (it_hbm, iidx_v, io_hbm)):
        for c in range(NCHK):
            s = c % 2
            if writes[s] is not None:
                writes[s].wait()
            gather = pltpu.make_async_copy(
                tbl.at[idx_v.at[pl.ds(c * CH, CH)]], bufs[s], gsems[s])
            gather.start()
            gather.wait()
            wr = pltpu.make_async_copy(
                bufs[s], out.at[pl.ds(base + c * CH, CH)], wsems[s])
            wr.start()
            writes[s] = wr
    for wr in writes:
        if wr is not None:
            wr.wait()


@functools.cache
def _sc_gather():
    mesh = plsc.VectorSubcoreMesh(core_axis_name="c", subcore_axis_name="s",
                                  num_cores=NC, num_subcores=NS)
    return pl.kernel(
        _sc_gather_body,
        out_type=(
            jax.ShapeDtypeStruct((B, PW), jnp.int32),
            jax.ShapeDtypeStruct((B, PW), jnp.int32),
        ),
        mesh=mesh,
        scratch_types=[
            pltpu.VMEM((BPW,), jnp.int32),
            pltpu.VMEM((BPW,), jnp.int32),
            pltpu.VMEM((CH, PW), jnp.int32),
            pltpu.VMEM((CH, PW), jnp.int32),
            pltpu.SemaphoreType.DMA,
            pltpu.SemaphoreType.DMA,
            pltpu.SemaphoreType.DMA,
            pltpu.SemaphoreType.DMA,
        ],
        compiler_params=pltpu.CompilerParams(use_tc_tiling_on_sc=True),
    )


BLK = 1024


def _unpack_quarter(pak_i32, q_i32):
    lo = lax.bitcast_convert_type(pak_i32 << 16, jnp.float32)
    hi = lax.bitcast_convert_type(pak_i32 & jnp.int32(-65536), jnp.float32)
    full = jnp.concatenate([lo, hi], axis=1)
    grp = jax.lax.broadcasted_iota(jnp.int32, (1, 2 * PW), 1)
    mask = ((grp >> 5) & 3) == q_i32
    return jnp.where(mask, full, 0.0)


def _mlp_body(u_ref, i_ref, uq_ref, iq_ref, gidx_ref,
              w1u_ref, w1i_ref, gt_ref, w1g_ref,
              b1_ref, w2_ref, b2_ref, w3_ref, b3_ref, w4_ref, b4_ref, o_ref):
    g_eff = gt_ref[...] @ w1g_ref[...] + b1_ref[...]
    gsel = jnp.where(gidx_ref[...] == 0, g_eff[0:1, :], g_eff[1:2, :])
    usel = _unpack_quarter(u_ref[...], uq_ref[...])
    isel = _unpack_quarter(i_ref[...], iq_ref[...])
    h = usel @ w1u_ref[...] + isel @ w1i_ref[...] + gsel
    h = jnp.maximum(h, 0.0)
    h = jnp.maximum(h @ w2_ref[...] + b2_ref[...], 0.0)
    h = jnp.maximum(h @ w3_ref[...] + b3_ref[...], 0.0)
    o_ref[...] = jnp.sum(h * w4_ref[...], axis=1) + b4_ref[0, 0]


def _mlp(u, i, uq, iq, gidx, w1u, w1i, gt, w1g, b1, w2, b2, w3, b3, w4row, b4):
    grid = (B // BLK,)
    full = lambda shape: pl.BlockSpec(shape, lambda n: (0, 0))
    return pl.pallas_call(
        _mlp_body,
        grid=grid,
        in_specs=[
            pl.BlockSpec((BLK, PW), lambda n: (n, 0)),
            pl.BlockSpec((BLK, PW), lambda n: (n, 0)),
            pl.BlockSpec((BLK, 1), lambda n: (n, 0)),
            pl.BlockSpec((BLK, 1), lambda n: (n, 0)),
            pl.BlockSpec((BLK, 1), lambda n: (n, 0)),
            full((2 * PW, 128)),
            full((2 * PW, 128)),
            full((2, DG)),
            full((DG, 128)),
            full((1, 128)),
            full((128, 64)),
            full((1, 64)),
            full((64, 32)),
            full((1, 32)),
            full((1, 32)),
            full((1, 1)),
        ],
        out_specs=pl.BlockSpec((BLK,), lambda n: (n,)),
        out_shape=jax.ShapeDtypeStruct((B,), jnp.float32),
    )(u, i, uq, iq, gidx, w1u, w1i, gt, w1g, b1, w2, b2, w3, b3, w4row, b4)


def _pack_table(tbl):
    ubits = lax.bitcast_convert_type(tbl, jnp.uint32)
    rne = (ubits + jnp.uint32(0x7FFF) + ((ubits >> 16) & jnp.uint32(1))) >> 16
    chunks = [rne[q::4, :DU // 2] | (rne[q::4, DU // 2:] << 16)
              for q in range(4)]
    return lax.bitcast_convert_type(jnp.concatenate(chunks, axis=1), jnp.int32)


def _stack4(w_block):
    lo4 = jnp.concatenate([w_block[:DU // 2]] * 4, axis=0)
    hi4 = jnp.concatenate([w_block[DU // 2:]] * 4, axis=0)
    return jnp.concatenate([lo4, hi4], axis=0)


def kernel(user_idx, item_idx, gender_idx, user_table, item_table,
           gender_table, W1, b1, W2, b2, W3, b3, W4, b4):
    uidx = user_idx.astype(jnp.int32)
    iidx = item_idx.astype(jnp.int32)
    gidx = gender_idx.astype(jnp.int32).reshape(B, 1)
    upak, ipak = _sc_gather()(uidx >> 2, iidx >> 2, _pack_table(user_table),
                              _pack_table(item_table))
    out = _mlp(
        upak, ipak, (uidx & 3).reshape(B, 1), (iidx & 3).reshape(B, 1), gidx,
        _stack4(W1[:DU]), _stack4(W1[DU:2 * DU]),
        gender_table, W1[2 * DU:], b1.reshape(1, 128),
        W2, b2.reshape(1, 64), W3, b3.reshape(1, 32),
        W4.reshape(1, DG), b4.reshape(1, 1),
    )
    return out


# speedup vs baseline: 15.0992x; 15.0992x over previous
import functools

import jax
import jax.numpy as jnp
from jax import lax
from jax.experimental import pallas as pl
from jax.experimental.pallas import tpu as pltpu
from jax.experimental.pallas import tpu_sc as plsc

B = 16384
DU = 64
DG = 32
NC = 2
NS = 16
NW = NC * NS
BPW = B // NW
CHK = 128
NCHK = BPW // CHK


def _sc_gather_body(uidx_hbm, iidx_hbm, ut_hbm, it_hbm,
                    uo_hbm, io_hbm,
                    uidx_v, iidx_v, ub0, ub1, ib0, ib1, gsem, osem):
    wid = lax.axis_index("s") * NC + lax.axis_index("c")
    base = wid * BPW
    pltpu.sync_copy(uidx_hbm.at[pl.ds(base, BPW)], uidx_v)
    pltpu.sync_copy(iidx_hbm.at[pl.ds(base, BPW)], iidx_v)

    ubufs = (ub0, ub1)
    ibufs = (ib0, ib1)
    pending = [None, None]

    for c in range(NCHK):
        slot = c % 2
        ubuf, ibuf = ubufs[slot], ibufs[slot]
        if pending[slot] is not None:
            for wb in pending[slot]:
                wb.wait()
            pending[slot] = None

        def enqueue(b, _):
            uv = uidx_v[pl.ds(c * CHK + b * 16, 16)]
            iv = iidx_v[pl.ds(c * CHK + b * 16, 16)]
            for k in range(16):
                ur = pl.multiple_of((uv[k] >> 1) * 2, 2)
                ir = pl.multiple_of((iv[k] >> 1) * 2, 2)
                pltpu.make_async_copy(
                    ut_hbm.at[pl.ds(ur, 2)],
                    ubuf.at[pl.ds((b * 16 + k) * 2, 2)], gsem).start()
                pltpu.make_async_copy(
                    it_hbm.at[pl.ds(ir, 2)],
                    ibuf.at[pl.ds((b * 16 + k) * 2, 2)], gsem).start()
            return _

        lax.fori_loop(0, CHK // 16, enqueue, 0)

        def drain(j, _):
            pltpu.make_async_copy(
                ut_hbm.at[pl.ds(0, 2)], ubuf.at[pl.ds(0, 2)], gsem).wait()
            pltpu.make_async_copy(
                it_hbm.at[pl.ds(0, 2)], ibuf.at[pl.ds(0, 2)], gsem).wait()
            return _

        lax.fori_loop(0, CHK, drain, 0)

        obase = 2 * (base + c * CHK)
        uwb = pltpu.make_async_copy(ubuf, uo_hbm.at[pl.ds(obase, 2 * CHK)], osem)
        iwb = pltpu.make_async_copy(ibuf, io_hbm.at[pl.ds(obase, 2 * CHK)], osem)
        uwb.start()
        iwb.start()
        pending[slot] = (uwb, iwb)

    for p in pending:
        if p is not None:
            for wb in p:
                wb.wait()


@functools.cache
def _sc_gather():
    mesh = plsc.VectorSubcoreMesh(core_axis_name="c", subcore_axis_name="s",
                                  num_cores=NC, num_subcores=NS)
    return pl.kernel(
        _sc_gather_body,
        out_type=(
            jax.ShapeDtypeStruct((2 * B, DU), jnp.bfloat16),
            jax.ShapeDtypeStruct((2 * B, DU), jnp.bfloat16),
        ),
        mesh=mesh,
        scratch_types=[
            pltpu.VMEM((BPW,), jnp.int32),
            pltpu.VMEM((BPW,), jnp.int32),
            pltpu.VMEM((2 * CHK, DU), jnp.bfloat16),
            pltpu.VMEM((2 * CHK, DU), jnp.bfloat16),
            pltpu.VMEM((2 * CHK, DU), jnp.bfloat16),
            pltpu.VMEM((2 * CHK, DU), jnp.bfloat16),
            pltpu.SemaphoreType.DMA,
            pltpu.SemaphoreType.DMA,
        ],
        compiler_params=pltpu.CompilerParams(use_tc_tiling_on_sc=True),
    )


BLK = 1024


def _mlp_body(u_ref, i_ref, uq_ref, iq_ref, gidx_ref,
              w1u_ref, w1i_ref, gt_ref, w1g_ref,
              b1_ref, w2_ref, b2_ref, w3_ref, b3_ref, w4_ref, b4_ref, o_ref):
    g_eff = gt_ref[...] @ w1g_ref[...] + b1_ref[...]
    gsel = jnp.where(gidx_ref[...] == 0, g_eff[0:1, :], g_eff[1:2, :])
    grp = jax.lax.broadcasted_iota(jnp.int32, (1, 2 * DU), 1) >> 6
    usel = jnp.where(grp == uq_ref[...], u_ref[...], 0).astype(jnp.float32)
    isel = jnp.where(grp == iq_ref[...], i_ref[...], 0).astype(jnp.float32)
    h = usel @ w1u_ref[...] + isel @ w1i_ref[...] + gsel
    h = jnp.maximum(h, 0.0)
    h = jnp.maximum(h @ w2_ref[...] + b2_ref[...], 0.0)
    h = jnp.maximum(h @ w3_ref[...] + b3_ref[...], 0.0)
    o_ref[...] = jnp.sum(h * w4_ref[...], axis=1) + b4_ref[0, 0]


def _mlp(u, i, uq, iq, gidx, w1u, w1i, gt, w1g, b1, w2, b2, w3, b3,
         w4row, b4):
    grid = (B // BLK,)
    full = lambda shape: pl.BlockSpec(shape, lambda n: (0, 0))
    return pl.pallas_call(
        _mlp_body,
        grid=grid,
        in_specs=[
            pl.BlockSpec((BLK, 2 * DU), lambda n: (n, 0)),
            pl.BlockSpec((BLK, 2 * DU), lambda n: (n, 0)),
            pl.BlockSpec((BLK, 1), lambda n: (n, 0)),
            pl.BlockSpec((BLK, 1), lambda n: (n, 0)),
            pl.BlockSpec((BLK, 1), lambda n: (n, 0)),
            full((2 * DU, 128)),
            full((2 * DU, 128)),
            full((2, DG)),
            full((DG, 128)),
            full((1, 128)),
            full((128, 64)),
            full((1, 64)),
            full((64, 32)),
            full((1, 32)),
            full((1, 32)),
            full((1, 1)),
        ],
        out_specs=pl.BlockSpec((BLK,), lambda n: (n,)),
        out_shape=jax.ShapeDtypeStruct((B,), jnp.float32),
    )(u, i, uq, iq, gidx, w1u, w1i, gt, w1g, b1, w2, b2, w3, b3, w4row, b4)


def kernel(user_idx, item_idx, gender_idx, user_table, item_table,
           gender_table, W1, b1, W2, b2, W3, b3, W4, b4):
    uidx = user_idx.astype(jnp.int32)
    iidx = item_idx.astype(jnp.int32)
    gidx = gender_idx.astype(jnp.int32).reshape(B, 1)
    u2, i2 = _sc_gather()(uidx, iidx, user_table.astype(jnp.bfloat16),
                          item_table.astype(jnp.bfloat16))
    out = _mlp(
        u2.reshape(B, 2 * DU), i2.reshape(B, 2 * DU),
        (uidx & 1).reshape(B, 1), (iidx & 1).reshape(B, 1), gidx,
        jnp.concatenate([W1[:DU]] * 2, axis=0),
        jnp.concatenate([W1[DU:2 * DU]] * 2, axis=0),
        gender_table, W1[2 * DU:], b1.reshape(1, 128),
        W2, b2.reshape(1, 64), W3, b3.reshape(1, 32),
        W4.reshape(1, DG), b4.reshape(1, 1),
    )
    return out
